# baseline (speedup 1.0000x reference)
import functools

import jax
import jax.numpy as jnp
from jax import lax
from jax.experimental import pallas as pl
from jax.experimental.pallas import tpu as pltpu
from jax.experimental.pallas import tpu_sc as plsc

Q = 32
D = 64
N = 1_000_000
BLK = 8192
NB = (N + BLK - 1) // BLK
NPAD = NB * BLK
NCHUNK = NB * 128
TOP_K = 10
CAND = TOP_K * (BLK // 128)
TEMPERATURE = 0.07
NEG = float("-inf")

NUM_SC_CORES = 2
NUM_SC_SUBCORES = 16
NUM_TILES = NUM_SC_CORES * NUM_SC_SUBCORES

_SC_PARAMS = pltpu.CompilerParams(needs_layout_passes=False)


def _sc_mesh():
    return plsc.VectorSubcoreMesh(core_axis_name="c", subcore_axis_name="s")


def _extract(vec, lane, j, fill):
    return jnp.max(jnp.where(lane == j, vec, fill))


def _k1_body(q_ref, k_ref, sims_ref, cmax_ref):
    b = pl.program_id(0)
    q = q_ref[...]
    qn = q / (jnp.sqrt(jnp.sum(q * q, axis=1, keepdims=True)) + 1e-8)
    qh = qn.astype(jnp.bfloat16)

    kb = k_ref[...]
    ksq = kb * kb
    s2 = jnp.sum(ksq.T, axis=0, keepdims=True)
    s2col = s2.reshape(BLK, 1)
    kh = (kb / (jnp.sqrt(s2col) + 1e-8)).astype(jnp.bfloat16)
    sim = lax.dot_general(qh, kh, (((1,), (1,)), ((), ())),
                          preferred_element_type=jnp.float32)

    col = b * BLK + lax.broadcasted_iota(jnp.int32, (1, BLK), 1)
    sim = jnp.where(col < N, sim, NEG)
    sims_ref[...] = sim

    m = sim[:, 0:128]
    for c in range(1, BLK // 128):
        m = jnp.maximum(m, sim[:, c * 128:(c + 1) * 128])
    cmax_ref[...] = m


def _run_k1(queries, keys):
    return pl.pallas_call(
        _k1_body,
        grid=(NB,),
        in_specs=[
            pl.BlockSpec((Q, D), lambda b: (0, 0)),
            pl.BlockSpec((BLK, D), lambda b: (b, 0)),
        ],
        out_specs=[
            pl.BlockSpec((Q, BLK), lambda b: (0, b)),
            pl.BlockSpec((Q, 128), lambda b: (0, b)),
        ],
        out_shape=[
            jax.ShapeDtypeStruct((Q, NPAD), jnp.float32),
            jax.ShapeDtypeStruct((Q, NCHUNK), jnp.float32),
        ],
    )(queries, keys)


def _k2_body(cmax_ref, bq_ref, lq_ref, pos_ref):
    cm = cmax_ref[...]
    ids = lax.broadcasted_iota(jnp.int32, (Q, NCHUNK), 1)
    io64 = lax.broadcasted_iota(jnp.int32, (Q, 64), 1) * 128
    bq_ref[...] = jnp.zeros((Q, 16), jnp.int32)
    lq_ref[...] = jnp.zeros((Q, 16), jnp.int32)
    for r in range(TOP_K):
        mval = jnp.max(cm, axis=1, keepdims=True)
        sel = jnp.min(jnp.where(cm == mval, ids, jnp.int32(2 ** 30)),
                      axis=1, keepdims=True)
        cm = jnp.where(ids == sel, NEG, cm)
        b = sel >> 7
        l = sel & 127
        bq_ref[:, r:r + 1] = b
        lq_ref[:, r:r + 1] = l
        pos_ref[:, r * 64:(r + 1) * 64] = b * BLK + l + io64


def _run_k2(cmax):
    return pl.pallas_call(
        _k2_body,
        out_shape=[
            jax.ShapeDtypeStruct((Q, 16), jnp.int32),
            jax.ShapeDtypeStruct((Q, 16), jnp.int32),
            jax.ShapeDtypeStruct((Q, CAND), jnp.int32),
        ],
    )(cmax)


def _run_k3(sims2d, bq16, lq16):
    rows_per_blk = BLK // 128
    row_stride = NPAD // 128

    @functools.partial(
        pl.kernel,
        out_type=jax.ShapeDtypeStruct((Q * CAND,), jnp.float32),
        mesh=_sc_mesh(),
        scratch_types=[
            pltpu.VMEM((16,), jnp.int32),
            pltpu.VMEM((16,), jnp.int32),
            pltpu.VMEM((TOP_K, rows_per_blk, 128), jnp.float32),
            pltpu.VMEM((CAND,), jnp.float32),
            pltpu.SemaphoreType.DMA,
        ],
        compiler_params=_SC_PARAMS,
    )
    def k3(sims_hbm, bq_hbm, lq_hbm, out_hbm, b_v, l_v, slabs_v, out_v, sem):
        q = lax.axis_index("s") * NUM_SC_CORES + lax.axis_index("c")
        pltpu.sync_copy(bq_hbm.at[q], b_v)
        pltpu.sync_copy(lq_hbm.at[q], l_v)
        bvec = b_v[...]
        lvec = l_v[...]
        lane = lax.iota(jnp.int32, 16)
        copies = []
        for r in range(TOP_K):
            br = _extract(bvec, lane, r, -1)
            base = q * row_stride + br * rows_per_blk
            copies.append(pltpu.async_copy(
                sims_hbm.at[pl.ds(base, rows_per_blk)], slabs_v.at[r], sem))
        for cp in copies:
            cp.wait()
        for r in range(TOP_K):
            lr = _extract(lvec, lane, r, -1)
            col = jnp.full((16,), lr, jnp.int32)
            for c4 in range(rows_per_blk // 16):
                rows = lane + 16 * c4
                vals = plsc.load_gather(slabs_v.at[r], [rows, col])
                out_v[pl.ds(r * 64 + c4 * 16, 16)] = vals
        pltpu.sync_copy(out_v, out_hbm.at[pl.ds(q * CAND, CAND)])

    return k3(sims2d, bq16, lq16)


def _k4_body(cs_ref, cp_ref, idx_ref, w_ref, vals_ref):
    cs = cs_ref[...]
    cp = cp_ref[...]
    vals_ref[...] = jnp.full((Q, 16), NEG, jnp.float32)
    idx_ref[...] = jnp.zeros((Q, 16), jnp.int32)
    for r in range(TOP_K):
        mval = jnp.max(cs, axis=1, keepdims=True)
        pi = jnp.min(jnp.where(cs == mval, cp, jnp.int32(2 ** 30)),
                     axis=1, keepdims=True)
        cs = jnp.where(cp == pi, NEG, cs)
        idx_ref[:, r:r + 1] = pi
        vals_ref[:, r:r + 1] = mval
    v = vals_ref[...] / TEMPERATURE
    e = jnp.exp(v - v[:, 0:1])
    w_ref[...] = e / jnp.sum(e, axis=1, keepdims=True)


def _run_k4(cand_sims, cand_pos):
    return pl.pallas_call(
        _k4_body,
        out_shape=[
            jax.ShapeDtypeStruct((Q, 16), jnp.int32),
            jax.ShapeDtypeStruct((Q, 16), jnp.float32),
        ],
        scratch_shapes=[pltpu.VMEM((Q, 16), jnp.float32)],
    )(cand_sims, cand_pos)


def _run_k5(tok2, idx16, w16):
    @functools.partial(
        pl.kernel,
        out_type=jax.ShapeDtypeStruct((Q, D), jnp.float32),
        mesh=_sc_mesh(),
        scratch_types=[
            pltpu.VMEM((16,), jnp.int32),
            pltpu.VMEM((16,), jnp.int32),
            pltpu.VMEM((16,), jnp.float32),
            pltpu.VMEM((16, 128), jnp.float32),
            pltpu.VMEM((D,), jnp.float32),
            pltpu.SemaphoreType.DMA,
        ],
        compiler_params=_SC_PARAMS,
    )
    def k5(tok_hbm, idx_hbm, w_hbm, out_hbm,
           idx_v, row_v, w_v, toks_v, acc_v, sem):
        q = lax.axis_index("s") * NUM_SC_CORES + lax.axis_index("c")
        pltpu.sync_copy(idx_hbm.at[q], idx_v)
        pltpu.sync_copy(w_hbm.at[q], w_v)
        ivec = idx_v[...]
        row_v[...] = ivec >> 1
        pltpu.async_copy(tok_hbm.at[row_v], toks_v, sem).wait()
        pvec = (ivec & 1).astype(jnp.float32)
        wvec = w_v[...]
        lane = lax.iota(jnp.int32, 16)
        for c in range(D // 16):
            acc = jnp.zeros((16,), jnp.float32)
            for j in range(TOP_K):
                wj = _extract(wvec, lane, j, NEG)
                pj = _extract(pvec, lane, j, NEG)
                lo = toks_v[j, pl.ds(c * 16, 16)]
                hi = toks_v[j, pl.ds(64 + c * 16, 16)]
                acc = acc + (lo * (1.0 - pj) + hi * pj) * wj
            acc_v[pl.ds(c * 16, 16)] = acc
        pltpu.sync_copy(acc_v, out_hbm.at[q])

    return k5(tok2, idx16, w16)


def kernel(queries, keys, token_bank):
    sims, cmax = _run_k1(queries, keys)
    bq16, lq16, cand_pos = _run_k2(cmax)
    cand = _run_k3(sims.reshape(Q * NPAD // 128, 128), bq16, lq16)
    idx16, w16 = _run_k4(cand.reshape(Q, CAND), cand_pos)
    prompt = _run_k5(token_bank.reshape(N // 2, 2 * D), idx16, w16)
    top_idx = idx16[:, :TOP_K]
    return prompt, top_idx


# speedup vs baseline: 1.3474x; 1.3474x over previous
import functools

import jax
import jax.numpy as jnp
from jax import lax
from jax.experimental import pallas as pl
from jax.experimental.pallas import tpu as pltpu
from jax.experimental.pallas import tpu_sc as plsc

Q = 32
D = 64
N = 1_000_000
BLK = 8192
NB = (N + BLK - 1) // BLK
NPAD = NB * BLK
NCHUNK = NB * 128
TOP_K = 10
CAND = TOP_K * (BLK // 128)
TEMPERATURE = 0.07
NEG = float("-inf")

NUM_SC_CORES = 2
NUM_SC_SUBCORES = 16
NUM_TILES = NUM_SC_CORES * NUM_SC_SUBCORES

_SC_PARAMS = pltpu.CompilerParams(needs_layout_passes=False)


def _sc_mesh():
    return plsc.VectorSubcoreMesh(core_axis_name="c", subcore_axis_name="s")


def _extract(vec, lane, j, fill):
    return jnp.max(jnp.where(lane == j, vec, fill))


def _k1_body(q_ref, k_ref, sims_ref, cmax_ref):
    b = pl.program_id(0)
    q = q_ref[...]
    qn = q / (jnp.sqrt(jnp.sum(q * q, axis=1, keepdims=True)) + 1e-8)
    qh = qn.astype(jnp.bfloat16)

    kb = k_ref[...]
    ksq = kb * kb
    s2 = jnp.sum(ksq.T, axis=0, keepdims=True)
    s2col = s2.reshape(BLK, 1)
    kh = (kb / (jnp.sqrt(s2col) + 1e-8)).astype(jnp.bfloat16)
    sim = lax.dot_general(qh, kh, (((1,), (1,)), ((), ())),
                          preferred_element_type=jnp.float32)

    col = b * BLK + lax.broadcasted_iota(jnp.int32, (1, BLK), 1)
    sim = jnp.where(col < N, sim, NEG)
    sims_ref[...] = sim

    m = sim[:, 0:128]
    for c in range(1, BLK // 128):
        m = jnp.maximum(m, sim[:, c * 128:(c + 1) * 128])
    cmax_ref[...] = m


def _run_k1(queries, keys):
    return pl.pallas_call(
        _k1_body,
        grid=(NB,),
        in_specs=[
            pl.BlockSpec((Q, D), lambda b: (0, 0)),
            pl.BlockSpec((BLK, D), lambda b: (b, 0)),
        ],
        out_specs=[
            pl.BlockSpec((Q, BLK), lambda b: (0, b)),
            pl.BlockSpec((Q, 128), lambda b: (0, b)),
        ],
        out_shape=[
            jax.ShapeDtypeStruct((Q, NPAD), jnp.float32),
            jax.ShapeDtypeStruct((Q, NCHUNK), jnp.float32),
        ],
    )(queries, keys)


def _k2_body(cmax_ref, bq_ref, lq_ref, pos_ref):
    cm = cmax_ref[...]
    ids = lax.broadcasted_iota(jnp.int32, (Q, NCHUNK), 1)
    io64 = lax.broadcasted_iota(jnp.int32, (Q, 64), 1) * 128
    bq_ref[...] = jnp.zeros((Q, 16), jnp.int32)
    lq_ref[...] = jnp.zeros((Q, 16), jnp.int32)
    for r in range(TOP_K):
        mval = jnp.max(cm, axis=1, keepdims=True)
        sel = jnp.min(jnp.where(cm == mval, ids, jnp.int32(2 ** 30)),
                      axis=1, keepdims=True)
        cm = jnp.where(ids == sel, NEG, cm)
        b = sel >> 7
        l = sel & 127
        bq_ref[:, r:r + 1] = b
        lq_ref[:, r:r + 1] = l
        pos_ref[:, r * 64:(r + 1) * 64] = b * BLK + l + io64


def _run_k2(cmax):
    return pl.pallas_call(
        _k2_body,
        out_shape=[
            jax.ShapeDtypeStruct((Q, 16), jnp.int32),
            jax.ShapeDtypeStruct((Q, 16), jnp.int32),
            jax.ShapeDtypeStruct((Q, CAND), jnp.int32),
        ],
    )(cmax)


def _run_k3(sims, bq16, lq16):
    @functools.partial(
        pl.kernel,
        out_type=jax.ShapeDtypeStruct((Q * CAND,), jnp.float32),
        mesh=_sc_mesh(),
        scratch_types=[
            pltpu.VMEM((16,), jnp.int32),
            pltpu.VMEM((16,), jnp.int32),
            pltpu.VMEM((TOP_K * BLK,), jnp.float32),
            pltpu.VMEM((CAND,), jnp.float32),
            pltpu.SemaphoreType.DMA,
        ],
        compiler_params=_SC_PARAMS,
    )
    def k3(sims_hbm, bq_hbm, lq_hbm, out_hbm, b_v, l_v, rows_v, out_v, sem):
        q = lax.axis_index("s") * NUM_SC_CORES + lax.axis_index("c")
        pltpu.sync_copy(bq_hbm.at[q], b_v)
        pltpu.sync_copy(lq_hbm.at[q], l_v)
        bvec = b_v[...]
        lvec = l_v[...]
        lane = lax.iota(jnp.int32, 16)
        copies = []
        for r in range(TOP_K):
            br = _extract(bvec, lane, r, -1)
            copies.append(pltpu.async_copy(
                sims_hbm.at[q, pl.ds(br * BLK, BLK)],
                rows_v.at[pl.ds(r * BLK, BLK)], sem))
        for cp in copies:
            cp.wait()
        for r in range(TOP_K):
            lr = _extract(lvec, lane, r, -1)
            for c4 in range(4):
                pos = r * BLK + lr + 2048 * c4 + 128 * lane
                vals = plsc.load_gather(rows_v, [pos])
                out_v[pl.ds(r * 64 + c4 * 16, 16)] = vals
        pltpu.sync_copy(out_v, out_hbm.at[pl.ds(q * CAND, CAND)])

    return k3(sims, bq16, lq16)


def _k4_body(cs_ref, cp_ref, idx_ref, w_ref, vals_ref):
    cs = cs_ref[...]
    cp = cp_ref[...]
    vals_ref[...] = jnp.full((Q, 16), NEG, jnp.float32)
    idx_ref[...] = jnp.zeros((Q, 16), jnp.int32)
    for r in range(TOP_K):
        mval = jnp.max(cs, axis=1, keepdims=True)
        pi = jnp.min(jnp.where(cs == mval, cp, jnp.int32(2 ** 30)),
                     axis=1, keepdims=True)
        cs = jnp.where(cp == pi, NEG, cs)
        idx_ref[:, r:r + 1] = pi
        vals_ref[:, r:r + 1] = mval
    v = vals_ref[...] / TEMPERATURE
    e = jnp.exp(v - v[:, 0:1])
    w_ref[...] = e / jnp.sum(e, axis=1, keepdims=True)


def _run_k4(cand_sims, cand_pos):
    return pl.pallas_call(
        _k4_body,
        out_shape=[
            jax.ShapeDtypeStruct((Q, 16), jnp.int32),
            jax.ShapeDtypeStruct((Q, 16), jnp.float32),
        ],
        scratch_shapes=[pltpu.VMEM((Q, 16), jnp.float32)],
    )(cand_sims, cand_pos)


def _run_k5(token_bank, idx16, w16):
    @functools.partial(
        pl.kernel,
        out_type=jax.ShapeDtypeStruct((Q, D), jnp.float32),
        mesh=_sc_mesh(),
        scratch_types=[
            pltpu.VMEM((16,), jnp.int32),
            pltpu.VMEM((16,), jnp.float32),
            pltpu.VMEM((16, D), jnp.float32),
            pltpu.VMEM((D,), jnp.float32),
            pltpu.SemaphoreType.DMA,
        ],
        compiler_params=_SC_PARAMS,
    )
    def k5(tok_hbm, idx_hbm, w_hbm, out_hbm, idx_v, w_v, toks_v, acc_v, sem):
        q = lax.axis_index("s") * NUM_SC_CORES + lax.axis_index("c")
        pltpu.sync_copy(idx_hbm.at[q], idx_v)
        pltpu.sync_copy(w_hbm.at[q], w_v)
        ivec = idx_v[...]
        wvec = w_v[...]
        lane = lax.iota(jnp.int32, 16)
        copies = []
        for j in range(TOP_K):
            rj = _extract(ivec, lane, j, -1)
            copies.append(pltpu.async_copy(
                tok_hbm.at[rj], toks_v.at[j], sem))
        for cp in copies:
            cp.wait()
        for c in range(D // 16):
            acc = jnp.zeros((16,), jnp.float32)
            for j in range(TOP_K):
                wj = _extract(wvec, lane, j, NEG)
                acc = acc + toks_v[j, pl.ds(c * 16, 16)] * wj
            acc_v[pl.ds(c * 16, 16)] = acc
        pltpu.sync_copy(acc_v, out_hbm.at[q])

    return k5(token_bank, idx16, w16)


def kernel(queries, keys, token_bank):
    sims, cmax = _run_k1(queries, keys)
    bq16, lq16, cand_pos = _run_k2(cmax)
    cand = _run_k3(sims, bq16, lq16)
    idx16, w16 = _run_k4(cand.reshape(Q, CAND), cand_pos)
    prompt = _run_k5(token_bank, idx16, w16)
    top_idx = idx16[:, :TOP_K]
    return prompt, top_idx
